# baseline (speedup 1.0000x reference)
import jax
import jax.numpy as jnp
from jax.experimental import pallas as pl
from jax.experimental.pallas import tpu as pltpu

TOKENS = 32768
D_MODEL = 768
N_EXPERTS = 8
TOP_K = 2

BT = 2048


def _router_body(x_ref, wt_ref, s_ref, ew_ref, ei_ref):
    x = x_ref[...]
    wt = wt_ref[...]
    logits = jax.lax.dot_general(
        x, wt, (((1,), (0,)), ((), ())), preferred_element_type=jnp.float32
    )
    m = jnp.max(logits, axis=-1, keepdims=True)
    e = jnp.exp(logits - m)
    p = e / jnp.sum(e, axis=-1, keepdims=True)
    s_ref[...] = p

    neg = jnp.float32(-jnp.inf)
    m1 = jnp.full((BT, 1), neg, jnp.float32)
    m2 = jnp.full((BT, 1), neg, jnp.float32)
    i1 = jnp.zeros((BT, 1), jnp.int32)
    i2 = jnp.zeros((BT, 1), jnp.int32)
    for ei in range(N_EXPERTS):
        v = p[:, ei : ei + 1]
        ec = jnp.full((BT, 1), ei, jnp.int32)
        gt1 = v > m1
        gt2 = v > m2
        i2 = jnp.where(gt1, i1, jnp.where(gt2, ec, i2))
        m2 = jnp.where(gt1, m1, jnp.where(gt2, v, m2))
        i1 = jnp.where(gt1, ec, i1)
        m1 = jnp.where(gt1, v, m1)
    ew_ref[...] = jnp.concatenate([m1, m2], axis=1)
    ei_ref[...] = jnp.concatenate([i1, i2], axis=1)


def kernel(x, W):
    wt = W.T
    grid = (TOKENS // BT,)
    scores, ew, ei = pl.pallas_call(
        _router_body,
        grid=grid,
        in_specs=[
            pl.BlockSpec((BT, D_MODEL), lambda i: (i, 0)),
            pl.BlockSpec((D_MODEL, N_EXPERTS), lambda i: (0, 0)),
        ],
        out_specs=[
            pl.BlockSpec((BT, N_EXPERTS), lambda i: (i, 0)),
            pl.BlockSpec((BT, TOP_K), lambda i: (i, 0)),
            pl.BlockSpec((BT, TOP_K), lambda i: (i, 0)),
        ],
        out_shape=[
            jax.ShapeDtypeStruct((TOKENS, N_EXPERTS), jnp.float32),
            jax.ShapeDtypeStruct((TOKENS, TOP_K), jnp.float32),
            jax.ShapeDtypeStruct((TOKENS, TOP_K), jnp.int32),
        ],
        compiler_params=pltpu.CompilerParams(
            dimension_semantics=("arbitrary",),
        ),
    )(x, wt)
    return (scores, ew, ei)


# speedup vs baseline: 2.7645x; 2.7645x over previous
import jax
import jax.numpy as jnp
from jax.experimental import pallas as pl
from jax.experimental.pallas import tpu as pltpu

TOKENS = 32768
D_MODEL = 768
N_EXPERTS = 8
TOP_K = 2

BT = 2048


def _router_body(x_ref, w_ref, s_ref, ew_ref, ei_ref):
    x = x_ref[...]
    w = w_ref[...]
    lt = jax.lax.dot_general(
        w, x, (((1,), (1,)), ((), ())), preferred_element_type=jnp.float32
    )
    m = jnp.max(lt, axis=0, keepdims=True)
    e = jnp.exp(lt - m)
    p = e / jnp.sum(e, axis=0, keepdims=True)
    s_ref[...] = p.T

    neg = jnp.float32(-1.0)
    m1 = jnp.full((1, BT), neg, jnp.float32)
    m2 = jnp.full((1, BT), neg, jnp.float32)
    i1 = jnp.zeros((1, BT), jnp.int32)
    i2 = jnp.zeros((1, BT), jnp.int32)
    for ei in range(N_EXPERTS):
        v = p[ei : ei + 1, :]
        ec = jnp.full((1, BT), ei, jnp.int32)
        gt1 = v > m1
        gt2 = v > m2
        i2 = jnp.where(gt1, i1, jnp.where(gt2, ec, i2))
        m2 = jnp.where(gt1, m1, jnp.where(gt2, v, m2))
        i1 = jnp.where(gt1, ec, i1)
        m1 = jnp.where(gt1, v, m1)
    ew_ref[...] = jnp.concatenate([m1, m2], axis=0).T
    ei_ref[...] = jnp.concatenate([i1, i2], axis=0).T


def kernel(x, W):
    grid = (TOKENS // BT,)
    scores, ew, ei = pl.pallas_call(
        _router_body,
        grid=grid,
        in_specs=[
            pl.BlockSpec((BT, D_MODEL), lambda i: (i, 0)),
            pl.BlockSpec((N_EXPERTS, D_MODEL), lambda i: (0, 0)),
        ],
        out_specs=[
            pl.BlockSpec((BT, N_EXPERTS), lambda i: (i, 0)),
            pl.BlockSpec((BT, TOP_K), lambda i: (i, 0)),
            pl.BlockSpec((BT, TOP_K), lambda i: (i, 0)),
        ],
        out_shape=[
            jax.ShapeDtypeStruct((TOKENS, N_EXPERTS), jnp.float32),
            jax.ShapeDtypeStruct((TOKENS, TOP_K), jnp.float32),
            jax.ShapeDtypeStruct((TOKENS, TOP_K), jnp.int32),
        ],
        compiler_params=pltpu.CompilerParams(
            dimension_semantics=("arbitrary",),
        ),
    )(x, W)
    return (scores, ew, ei)


# speedup vs baseline: 2.8678x; 1.0374x over previous
import jax
import jax.numpy as jnp
from jax.experimental import pallas as pl
from jax.experimental.pallas import tpu as pltpu

TOKENS = 32768
D_MODEL = 768
N_EXPERTS = 8
TOP_K = 2

BT = 4096


def _router_body(x_ref, w_ref, s_ref, ew_ref, ei_ref):
    x = x_ref[...]
    w = w_ref[...]
    lt = jax.lax.dot_general(
        w, x, (((1,), (1,)), ((), ())), preferred_element_type=jnp.float32
    )
    m = jnp.max(lt, axis=0, keepdims=True)
    e = jnp.exp(lt - m)
    p = e / jnp.sum(e, axis=0, keepdims=True)
    s_ref[...] = p.T

    neg = jnp.float32(-1.0)
    m1 = jnp.full((1, BT), neg, jnp.float32)
    m2 = jnp.full((1, BT), neg, jnp.float32)
    i1 = jnp.zeros((1, BT), jnp.int32)
    i2 = jnp.zeros((1, BT), jnp.int32)
    for ei in range(N_EXPERTS):
        v = p[ei : ei + 1, :]
        ec = jnp.full((1, BT), ei, jnp.int32)
        gt1 = v > m1
        gt2 = v > m2
        i2 = jnp.where(gt1, i1, jnp.where(gt2, ec, i2))
        m2 = jnp.where(gt1, m1, jnp.where(gt2, v, m2))
        i1 = jnp.where(gt1, ec, i1)
        m1 = jnp.where(gt1, v, m1)
    ew_ref[...] = jnp.concatenate([m1, m2], axis=0).T
    ei_ref[...] = jnp.concatenate([i1, i2], axis=0).T


def kernel(x, W):
    grid = (TOKENS // BT,)
    scores, ew, ei = pl.pallas_call(
        _router_body,
        grid=grid,
        in_specs=[
            pl.BlockSpec((BT, D_MODEL), lambda i: (i, 0)),
            pl.BlockSpec((N_EXPERTS, D_MODEL), lambda i: (0, 0)),
        ],
        out_specs=[
            pl.BlockSpec((BT, N_EXPERTS), lambda i: (i, 0)),
            pl.BlockSpec((BT, TOP_K), lambda i: (i, 0)),
            pl.BlockSpec((BT, TOP_K), lambda i: (i, 0)),
        ],
        out_shape=[
            jax.ShapeDtypeStruct((TOKENS, N_EXPERTS), jnp.float32),
            jax.ShapeDtypeStruct((TOKENS, TOP_K), jnp.float32),
            jax.ShapeDtypeStruct((TOKENS, TOP_K), jnp.int32),
        ],
        compiler_params=pltpu.CompilerParams(
            dimension_semantics=("arbitrary",),
        ),
    )(x, W)
    return (scores, ew, ei)
